# baseline (speedup 1.0000x reference)
import functools

import jax
import jax.numpy as jnp
from jax import lax
from jax.experimental import pallas as pl
from jax.experimental.pallas import tpu as pltpu
from jax.experimental.pallas import tpu_sc as plsc

S, N, D = 50, 4096, 512
T = S * N
NUM_SECTIONS = 5
SECTION = 128
R = NUM_SECTIONS * SECTION + 8
GRID = 128
PAIR = GRID * GRID
NSTREAM = 3
RC = SECTION + 2 * PAIR

NC, NS = 2, 16
NW = NC * NS
TPW = T // NW
C = 32
G = NSTREAM * C
NCHUNK = TPW // C


def _build_table_body(lt_ref, bt_ref, w_ref, b_ref, out_ref):
    lt = lt_ref[...]
    bt = bt_ref[...]
    w = w_ref[...]
    b = b_ref[...]
    dn = (((1,), (1,)), ((), ()))
    pieces = [
        lax.dot_general(lt, w[:, 0:SECTION], dn,
                        preferred_element_type=jnp.float32) + b
    ]
    for k in range(1, NUM_SECTIONS):
        pieces.append(
            lax.dot_general(bt, w[:, SECTION * k:SECTION * (k + 1)], dn,
                            preferred_element_type=jnp.float32))
    pieces.append(jnp.zeros((8, D), jnp.float32))
    out_ref[...] = jnp.concatenate(pieces, axis=0)


def _build_table(label_table, bbox_table, W, b2):
    lt_pad = jnp.zeros((SECTION, 128), jnp.float32).at[:label_table.shape[0]].set(label_table)
    return pl.pallas_call(
        _build_table_body,
        out_shape=jax.ShapeDtypeStruct((R, D), jnp.float32),
    )(lt_pad, bbox_table, W, b2)


def _build_combined_body(base_ref, out_ref):
    i = pl.program_id(0)
    is_xy = jnp.logical_and(i >= 1, i <= GRID)
    row_start = jnp.where(i == 0, NUM_SECTIONS * SECTION,
                          jnp.where(is_xy, SECTION + (i - 1),
                                    3 * SECTION + (i - 1 - GRID)))
    blk_start = jnp.where(i == 0, 0,
                          jnp.where(is_xy, 2 * SECTION, 4 * SECTION))
    row = base_ref[pl.ds(row_start, 1), :]
    blk = base_ref[pl.ds(blk_start, SECTION), :]
    out_ref[...] = row + blk


def _build_combined(base):
    return pl.pallas_call(
        _build_combined_body,
        grid=(2 * GRID + 1,),
        in_specs=[pl.BlockSpec((R, D), lambda i: (0, 0))],
        out_specs=pl.BlockSpec((SECTION, D), lambda i: (i, 0)),
        out_shape=jax.ShapeDtypeStruct((RC, D), jnp.float32),
    )(base)


def _sc_gather_reduce(streams, table):
    mesh = plsc.VectorSubcoreMesh(core_axis_name="c", subcore_axis_name="s")

    @functools.partial(
        pl.kernel,
        mesh=mesh,
        out_type=jax.ShapeDtypeStruct((T, D), jnp.float32),
        scratch_types=[
            pltpu.VMEM((2, G), jnp.int32),
            pltpu.VMEM((2, G, D), jnp.float32),
            pltpu.SemaphoreType.DMA,
            pltpu.SemaphoreType.DMA,
            pltpu.SemaphoreType.DMA,
            pltpu.SemaphoreType.DMA,
            pltpu.SemaphoreType.DMA,
            pltpu.SemaphoreType.DMA,
        ],
    )
    def k(streams_hbm, tab_hbm, out_hbm, idx_v, rows_v,
          isem0, isem1, gsem0, gsem1, osem0, osem1):
        isems = (isem0, isem1)
        gsems = (gsem0, gsem1)
        osems = (osem0, osem1)
        wid = lax.axis_index("s") * NC + lax.axis_index("c")
        wbase = wid * TPW
        woff = wid * NCHUNK * G

        def fire_idx(ci, b):
            pltpu.async_copy(
                streams_hbm.at[pl.ds(woff + ci * G, G)], idx_v.at[b],
                isems[b])

        def wait_idx(b):
            pltpu.make_async_copy(
                streams_hbm.at[pl.ds(0, G)], idx_v.at[b], isems[b]).wait()

        def fire_gather(b):
            pltpu.async_copy(
                tab_hbm.at[idx_v.at[b]], rows_v.at[b], gsems[b])

        def wait_gather(b):
            pltpu.make_async_copy(
                tab_hbm.at[idx_v.at[b]], rows_v.at[b], gsems[b]).wait()

        def fire_out(ci, b):
            pltpu.async_copy(
                rows_v.at[b, pl.ds(0, C)],
                out_hbm.at[pl.ds(wbase + ci * C, C)], osems[b])

        def wait_out(b):
            pltpu.make_async_copy(
                rows_v.at[b, pl.ds(0, C)],
                out_hbm.at[pl.ds(0, C)], osems[b]).wait()

        def reduce_chunk(b):
            def row_body(r, rc):
                for j in range(D // 16):
                    sl = pl.ds(j * 16, 16)
                    v = rows_v[b, r, sl]
                    v = v + rows_v[b, C + r, sl]
                    v = v + rows_v[b, 2 * C + r, sl]
                    rows_v[b, r, sl] = v
                return rc
            lax.fori_loop(0, C, row_body, 0)

        fire_idx(0, 0)
        wait_idx(0)
        fire_gather(0)
        fire_idx(1, 1)

        def step(ci, carry):
            b = lax.rem(ci, 2)

            def half(bs):
                nbs = 1 - bs
                wait_gather(bs)

                @pl.when(ci + 1 < NCHUNK)
                def _():
                    wait_idx(nbs)
                    fire_gather(nbs)

                @pl.when(ci >= 2)
                def _():
                    wait_out(bs)

                reduce_chunk(bs)
                fire_out(ci, bs)

                @pl.when(ci + 2 < NCHUNK)
                def _():
                    fire_idx(ci + 2, bs)

            @pl.when(b == 0)
            def _():
                half(0)

            @pl.when(b == 1)
            def _():
                half(1)

            return carry

        lax.fori_loop(0, NCHUNK, step, 0)
        wait_out(0)
        wait_out(1)

    return k(streams, table)


def kernel(label, box, label_table, bbox_table, W, b):
    label = label.astype(jnp.int32)
    box = box.astype(jnp.int32)
    base = _build_table(label_table.astype(jnp.float32),
                        bbox_table.astype(jnp.float32),
                        W.astype(jnp.float32),
                        b.astype(jnp.float32).reshape(1, D))
    table = _build_combined(base)
    lab = label.reshape(T)
    bx = box.reshape(T, 4)
    streams = jnp.stack(
        [lab,
         SECTION + bx[:, 0] * GRID + bx[:, 1],
         SECTION + PAIR + bx[:, 2] * GRID + bx[:, 3]], axis=0)
    streams = streams.reshape(3, NW, NCHUNK, C).transpose(1, 2, 0, 3).reshape(-1)
    out = _sc_gather_reduce(streams, table)
    return out.reshape(S, N, D)
